# Initial kernel scaffold; baseline (speedup 1.0000x reference)
import functools

import jax
import jax.numpy as jnp
from jax import lax
from jax.experimental import pallas as pl
from jax.experimental.pallas import tpu as pltpu
from jax.experimental.pallas import tpu_sc as plsc

N = 10000
E = 320000
D = 128

NC = 2
NS = 16
NW = NC * NS
EPW = E // NW
B = 80
NBATCH = EPW // B
RPT = N // NS
ZR = 125

_mesh = plsc.VectorSubcoreMesh(core_axis_name="c", subcore_axis_name="s")


@functools.partial(
    pl.kernel,
    out_type=jax.ShapeDtypeStruct((NC, N, 16), jnp.float32),
    mesh=_mesh,
    scratch_types=[
        pltpu.VMEM((B,), jnp.int32),
        pltpu.VMEM((B, 16), jnp.float32),
        pltpu.VMEM((ZR, 16), jnp.float32),
        pltpu.VMEM_SHARED((N, 16), jnp.float32),
    ],
)
def _sc_hist(col_hbm, out_hbm, cidx, ones, zbuf, acc):
    c = lax.axis_index("c")
    s = lax.axis_index("s")
    w = s * NC + c

    lane = lax.iota(jnp.int32, 16)
    onehot = jnp.where(lane == 0, 1.0, 0.0).astype(jnp.float32)
    zero = jnp.zeros((16,), jnp.float32)

    def init(i, carry):
        ones[i, :] = onehot
        return carry

    lax.fori_loop(0, B, init, 0)

    def zinit(i, carry):
        zbuf[i, :] = zero
        return carry

    lax.fori_loop(0, ZR, zinit, 0)
    for k in range(RPT // ZR):
        pltpu.sync_copy(zbuf, acc.at[pl.ds(s * RPT + k * ZR, ZR)])
    plsc.subcore_barrier()

    def body(i, carry):
        base = pl.multiple_of(w * EPW + i * B, 8)
        pltpu.sync_copy(col_hbm.at[pl.ds(base, B)], cidx)
        pltpu.sync_copy(ones, acc.at[cidx], add=True)
        return carry

    lax.fori_loop(0, NBATCH, body, 0)
    plsc.subcore_barrier()
    pltpu.sync_copy(acc.at[pl.ds(s * RPT, RPT)],
                    out_hbm.at[c, pl.ds(s * RPT, RPT)])


@functools.partial(
    pl.kernel,
    out_type=jax.ShapeDtypeStruct((NC, N, D), jnp.float32),
    mesh=_mesh,
    scratch_types=[
        pltpu.VMEM((B,), jnp.int32),
        pltpu.VMEM((B,), jnp.int32),
        pltpu.VMEM((B, D), jnp.float32),
        pltpu.VMEM((ZR, D), jnp.float32),
        pltpu.VMEM_SHARED((N, D), jnp.float32),
    ],
)
def _sc_agg(hp_hbm, row_hbm, col_hbm, out_hbm, ridx, cidx, rows, zbuf, acc):
    c = lax.axis_index("c")
    s = lax.axis_index("s")
    w = s * NC + c

    zero = jnp.zeros((16,), jnp.float32)

    def zinit(i, carry):
        for j in range(D // 16):
            zbuf[i, pl.ds(j * 16, 16)] = zero
        return carry

    lax.fori_loop(0, ZR, zinit, 0)
    for k in range(RPT // ZR):
        pltpu.sync_copy(zbuf, acc.at[pl.ds(s * RPT + k * ZR, ZR)])
    plsc.subcore_barrier()

    def body(i, carry):
        base = pl.multiple_of(w * EPW + i * B, 8)
        pltpu.sync_copy(row_hbm.at[pl.ds(base, B)], ridx)
        pltpu.sync_copy(col_hbm.at[pl.ds(base, B)], cidx)
        pltpu.sync_copy(hp_hbm.at[ridx], rows)
        pltpu.sync_copy(rows, acc.at[cidx], add=True)
        return carry

    lax.fori_loop(0, NBATCH, body, 0)
    plsc.subcore_barrier()
    pltpu.sync_copy(acc.at[pl.ds(s * RPT, RPT)],
                    out_hbm.at[c, pl.ds(s * RPT, RPT)])


def _first_body(cnt_ref, x_ref, w_ref, h_ref, hp_ref, dinv_ref):
    deg = jnp.sum(cnt_ref[0] + cnt_ref[1], axis=1, keepdims=True) + 2.0
    dinv = lax.rsqrt(deg)
    h = jnp.dot(x_ref[...], w_ref[...], preferred_element_type=jnp.float32)
    h_ref[...] = h
    hp_ref[...] = h * dinv
    dinv_ref[...] = dinv


def _tc_first(cnt, x, W1):
    return pl.pallas_call(
        _first_body,
        out_shape=[
            jax.ShapeDtypeStruct((N, D), jnp.float32),
            jax.ShapeDtypeStruct((N, D), jnp.float32),
            jax.ShapeDtypeStruct((N, 1), jnp.float32),
        ],
    )(cnt, x, W1)


def _mid_body(agg_ref, hprev_ref, dinv_ref, b_ref, w_ref, h_ref, hp_ref):
    dinv = dinv_ref[...]
    z = (dinv * (agg_ref[0] + agg_ref[1])
         + (2.0 * dinv * dinv) * hprev_ref[...] + b_ref[...])
    h = jnp.dot(z, w_ref[...], preferred_element_type=jnp.float32)
    h_ref[...] = h
    hp_ref[...] = h * dinv


def _tc_mid(agg, hprev, dinv, b2d, W):
    return pl.pallas_call(
        _mid_body,
        out_shape=[
            jax.ShapeDtypeStruct((N, D), jnp.float32),
            jax.ShapeDtypeStruct((N, D), jnp.float32),
        ],
    )(agg, hprev, dinv, b2d, W)


def _final_body(agg_ref, hprev_ref, dinv_ref, b_ref, out_ref):
    dinv = dinv_ref[...]
    out_ref[...] = (dinv * (agg_ref[0] + agg_ref[1])
                    + (2.0 * dinv * dinv) * hprev_ref[...] + b_ref[...])


def _tc_final(agg, hprev, dinv, b2d):
    return pl.pallas_call(
        _final_body,
        out_shape=jax.ShapeDtypeStruct((N, D), jnp.float32),
    )(agg, hprev, dinv, b2d)


def kernel(x, edge_index, W1, b1, W2, b2):
    row = edge_index[0]
    col = edge_index[1]
    b1d = b1.reshape(1, D)
    b2d = b2.reshape(1, D)

    cnt = _sc_hist(col)
    h, hp, dinv = _tc_first(cnt, x, W1)
    for b in (b1d, b2d, b2d, b2d):
        agg = _sc_agg(hp, row, col)
        h, hp = _tc_mid(agg, h, dinv, b, W2)
    agg = _sc_agg(hp, row, col)
    return _tc_final(agg, h, dinv, b2d)


# speedup vs baseline: 9.3879x; 9.3879x over previous
import functools

import jax
import jax.numpy as jnp
from jax import lax
from jax.experimental import pallas as pl
from jax.experimental.pallas import tpu as pltpu
from jax.experimental.pallas import tpu_sc as plsc

N = 10000
E = 320000
D = 128

NC = 2
NS = 16
NW = NC * NS
EPW = E // NW
B = 80
NBATCH = EPW // B
RPT = 624
TAIL = N - NS * RPT
ZR = 104

_mesh = plsc.VectorSubcoreMesh(core_axis_name="c", subcore_axis_name="s")


@functools.partial(
    pl.kernel,
    out_type=jax.ShapeDtypeStruct((NC, N, D), jnp.float32),
    mesh=_mesh,
    scratch_types=[
        pltpu.VMEM((B,), jnp.int32),
        pltpu.VMEM((B, D), jnp.float32),
        pltpu.VMEM((ZR, D), jnp.float32),
        pltpu.VMEM_SHARED((N, D), jnp.float32),
    ],
)
def _sc_hist(col_hbm, out_hbm, cidx, ones, zbuf, acc):
    c = lax.axis_index("c")
    s = lax.axis_index("s")
    w = s * NC + c

    lane = lax.iota(jnp.int32, 16)
    onehot = jnp.where(lane == 0, 1.0, 0.0).astype(jnp.float32)
    zero = jnp.zeros((16,), jnp.float32)

    def init(i, carry):
        ones[i, pl.ds(0, 16)] = onehot
        for j in range(1, D // 16):
            ones[i, pl.ds(j * 16, 16)] = zero
        return carry

    lax.fori_loop(0, B, init, 0)

    def zinit(i, carry):
        for j in range(D // 16):
            zbuf[i, pl.ds(j * 16, 16)] = zero
        return carry

    lax.fori_loop(0, ZR, zinit, 0)
    rbase = pl.multiple_of(s * RPT, 8)
    for k in range(RPT // ZR):
        pltpu.sync_copy(zbuf, acc.at[pl.ds(rbase + k * ZR, ZR)])

    @pl.when(s == NS - 1)
    def _():
        pltpu.sync_copy(zbuf.at[pl.ds(0, TAIL)], acc.at[pl.ds(NS * RPT, TAIL)])

    plsc.subcore_barrier()

    def body(i, carry):
        base = pl.multiple_of(w * EPW + i * B, 8)
        pltpu.sync_copy(col_hbm.at[pl.ds(base, B)], cidx)
        pltpu.sync_copy(ones, acc.at[cidx], add=True)
        return carry

    lax.fori_loop(0, NBATCH, body, 0)
    plsc.subcore_barrier()
    pltpu.sync_copy(acc.at[pl.ds(rbase, RPT)],
                    out_hbm.at[c, pl.ds(rbase, RPT)])

    @pl.when(s == NS - 1)
    def _():
        pltpu.sync_copy(acc.at[pl.ds(NS * RPT, TAIL)],
                        out_hbm.at[c, pl.ds(NS * RPT, TAIL)])


@functools.partial(
    pl.kernel,
    out_type=jax.ShapeDtypeStruct((NC, N, D), jnp.float32),
    mesh=_mesh,
    scratch_types=[
        pltpu.VMEM((B,), jnp.int32),
        pltpu.VMEM((B,), jnp.int32),
        pltpu.VMEM((B, D), jnp.float32),
        pltpu.VMEM((ZR, D), jnp.float32),
        pltpu.VMEM_SHARED((N, D), jnp.float32),
    ],
)
def _sc_agg(hp_hbm, row_hbm, col_hbm, out_hbm, ridx, cidx, rows, zbuf, acc):
    c = lax.axis_index("c")
    s = lax.axis_index("s")
    w = s * NC + c

    zero = jnp.zeros((16,), jnp.float32)

    def zinit(i, carry):
        for j in range(D // 16):
            zbuf[i, pl.ds(j * 16, 16)] = zero
        return carry

    lax.fori_loop(0, ZR, zinit, 0)
    rbase = pl.multiple_of(s * RPT, 8)
    for k in range(RPT // ZR):
        pltpu.sync_copy(zbuf, acc.at[pl.ds(rbase + k * ZR, ZR)])

    @pl.when(s == NS - 1)
    def _():
        pltpu.sync_copy(zbuf.at[pl.ds(0, TAIL)], acc.at[pl.ds(NS * RPT, TAIL)])

    plsc.subcore_barrier()

    def body(i, carry):
        base = pl.multiple_of(w * EPW + i * B, 8)
        pltpu.sync_copy(row_hbm.at[pl.ds(base, B)], ridx)
        pltpu.sync_copy(col_hbm.at[pl.ds(base, B)], cidx)
        pltpu.sync_copy(hp_hbm.at[ridx], rows)
        pltpu.sync_copy(rows, acc.at[cidx], add=True)
        return carry

    lax.fori_loop(0, NBATCH, body, 0)
    plsc.subcore_barrier()
    pltpu.sync_copy(acc.at[pl.ds(rbase, RPT)],
                    out_hbm.at[c, pl.ds(rbase, RPT)])

    @pl.when(s == NS - 1)
    def _():
        pltpu.sync_copy(acc.at[pl.ds(NS * RPT, TAIL)],
                        out_hbm.at[c, pl.ds(NS * RPT, TAIL)])


def _first_body(cnt_ref, x_ref, w_ref, h_ref, hp_ref, dinv_ref):
    deg = jnp.sum(cnt_ref[0] + cnt_ref[1], axis=1, keepdims=True) + 2.0
    dinv = lax.rsqrt(deg)
    h = jnp.dot(x_ref[...], w_ref[...], preferred_element_type=jnp.float32)
    h_ref[...] = h
    hp_ref[...] = h * dinv
    dinv_ref[...] = dinv


def _tc_first(cnt, x, W1):
    return pl.pallas_call(
        _first_body,
        out_shape=[
            jax.ShapeDtypeStruct((N, D), jnp.float32),
            jax.ShapeDtypeStruct((N, D), jnp.float32),
            jax.ShapeDtypeStruct((N, 1), jnp.float32),
        ],
    )(cnt, x, W1)


def _mid_body(agg_ref, hprev_ref, dinv_ref, b_ref, w_ref, h_ref, hp_ref):
    dinv = dinv_ref[...]
    z = (dinv * (agg_ref[0] + agg_ref[1])
         + (2.0 * dinv * dinv) * hprev_ref[...] + b_ref[...])
    h = jnp.dot(z, w_ref[...], preferred_element_type=jnp.float32)
    h_ref[...] = h
    hp_ref[...] = h * dinv


def _tc_mid(agg, hprev, dinv, b2d, W):
    return pl.pallas_call(
        _mid_body,
        out_shape=[
            jax.ShapeDtypeStruct((N, D), jnp.float32),
            jax.ShapeDtypeStruct((N, D), jnp.float32),
        ],
    )(agg, hprev, dinv, b2d, W)


def _final_body(agg_ref, hprev_ref, dinv_ref, b_ref, out_ref):
    dinv = dinv_ref[...]
    out_ref[...] = (dinv * (agg_ref[0] + agg_ref[1])
                    + (2.0 * dinv * dinv) * hprev_ref[...] + b_ref[...])


def _tc_final(agg, hprev, dinv, b2d):
    return pl.pallas_call(
        _final_body,
        out_shape=jax.ShapeDtypeStruct((N, D), jnp.float32),
    )(agg, hprev, dinv, b2d)


def kernel(x, edge_index, W1, b1, W2, b2):
    row = edge_index[0]
    col = edge_index[1]
    b1d = b1.reshape(1, D)
    b2d = b2.reshape(1, D)

    cnt = _sc_hist(col)
    h, hp, dinv = _tc_first(cnt, x, W1)
    for b in (b1d, b2d, b2d, b2d):
        agg = _sc_agg(hp, row, col)
        h, hp = _tc_mid(agg, h, dinv, b, W2)
    agg = _sc_agg(hp, row, col)
    return _tc_final(agg, h, dinv, b2d)


# speedup vs baseline: 17.4662x; 1.8605x over previous
import functools

import jax
import jax.numpy as jnp
from jax import lax
from jax.experimental import pallas as pl
from jax.experimental.pallas import tpu as pltpu
from jax.experimental.pallas import tpu_sc as plsc

N = 10000
E = 320000
D = 128

NC = 2
NS = 16
NW = NC * NS
BB = 128
GB = E // BB
WB = GB // NW
WX = GB - WB * NW
TPAIR = WB // 2
SLAB = WB * BB
XOFF = NW * SLAB
RPT = 624
TAIL = N - NS * RPT

_mesh = plsc.VectorSubcoreMesh(core_axis_name="c", subcore_axis_name="s")


@functools.partial(
    pl.kernel,
    out_type=jax.ShapeDtypeStruct((NC, N, D), jnp.float32),
    mesh=_mesh,
    scratch_types=[
        pltpu.VMEM((WB + 1, BB), jnp.int32),
        pltpu.VMEM((BB, D), jnp.float32),
        pltpu.VMEM_SHARED((N, D), jnp.float32),
        pltpu.SemaphoreType.DMA,
        pltpu.SemaphoreType.DMA,
        pltpu.SemaphoreType.DMA,
    ],
)
def _sc_hist(col_hbm, zero_hbm, out_hbm, cidx, ones, acc, sem_z, sem_i,
             sem_s):
    c = lax.axis_index("c")
    s = lax.axis_index("s")
    w = s * NC + c
    xtra = w < WX
    nb = WB + jnp.where(xtra, 1, 0)
    rbase = pl.multiple_of(s * RPT, 8)
    eoff = w * SLAB

    dz = pltpu.async_copy(zero_hbm, acc.at[pl.ds(rbase, RPT)], sem_z)

    @pl.when(s == NS - 1)
    def _():
        pltpu.async_copy(zero_hbm.at[pl.ds(0, TAIL)],
                         acc.at[pl.ds(NS * RPT, TAIL)], sem_z)

    def iload(i, carry):
        pltpu.async_copy(col_hbm.at[pl.ds(eoff + i * BB, BB)], cidx.at[i],
                         sem_i)
        return carry

    lax.fori_loop(0, WB, iload, 0)

    @pl.when(xtra)
    def _():
        pltpu.async_copy(col_hbm.at[pl.ds(XOFF + w * BB, BB)], cidx.at[WB],
                         sem_i)

    lane = lax.iota(jnp.int32, 16)
    onehot = jnp.where(lane == 0, 1.0, 0.0).astype(jnp.float32)
    zero = jnp.zeros((16,), jnp.float32)

    def init(i, carry):
        ones[i, pl.ds(0, 16)] = onehot
        for j in range(1, D // 16):
            ones[i, pl.ds(j * 16, 16)] = zero
        return carry

    lax.fori_loop(0, BB, init, 0)

    def idrain(i, carry):
        pltpu.make_async_copy(col_hbm.at[pl.ds(eoff, BB)], cidx.at[0],
                              sem_i).wait()
        return carry

    lax.fori_loop(0, nb, idrain, 0)
    dz.wait()

    @pl.when(s == NS - 1)
    def _():
        pltpu.make_async_copy(zero_hbm.at[pl.ds(0, TAIL)],
                              acc.at[pl.ds(NS * RPT, TAIL)], sem_z).wait()

    plsc.subcore_barrier()

    def body(i, carry):
        pltpu.async_copy(ones, acc.at[cidx.at[i]], sem_s, add=True)

        @pl.when(i >= 2)
        def _():
            pltpu.make_async_copy(ones, acc.at[cidx.at[i - 2]], sem_s).wait()

        return carry

    lax.fori_loop(0, nb, body, 0)

    def drain(i, carry):
        pltpu.make_async_copy(ones, acc.at[cidx.at[nb - 2 + i]], sem_s).wait()
        return carry

    lax.fori_loop(0, 2, drain, 0)
    plsc.subcore_barrier()
    pltpu.sync_copy(acc.at[pl.ds(rbase, RPT)],
                    out_hbm.at[c, pl.ds(rbase, RPT)])

    @pl.when(s == NS - 1)
    def _():
        pltpu.sync_copy(acc.at[pl.ds(NS * RPT, TAIL)],
                        out_hbm.at[c, pl.ds(NS * RPT, TAIL)])


@functools.partial(
    pl.kernel,
    out_type=jax.ShapeDtypeStruct((NC, N, D), jnp.float32),
    mesh=_mesh,
    scratch_types=[
        pltpu.VMEM((2, BB), jnp.int32),
        pltpu.VMEM((2, BB), jnp.int32),
        pltpu.VMEM((BB, D), jnp.float32),
        pltpu.VMEM((BB, D), jnp.float32),
        pltpu.VMEM_SHARED((N, D), jnp.float32),
        pltpu.SemaphoreType.DMA,
        pltpu.SemaphoreType.DMA,
        pltpu.SemaphoreType.DMA,
        pltpu.SemaphoreType.DMA,
        pltpu.SemaphoreType.DMA,
        pltpu.SemaphoreType.DMA,
        pltpu.SemaphoreType.DMA,
    ],
)
def _sc_agg(hp_hbm, row_hbm, col_hbm, zero_hbm, out_hbm,
            idxA, idxB, rowsA, rowsB, acc,
            sem_z, sem_ia, sem_ib, sem_ga, sem_gb, sem_sa, sem_sb):
    c = lax.axis_index("c")
    s = lax.axis_index("s")
    w = s * NC + c
    rbase = pl.multiple_of(s * RPT, 8)
    eoff = w * SLAB

    dz = pltpu.async_copy(zero_hbm, acc.at[pl.ds(rbase, RPT)], sem_z)

    @pl.when(s == NS - 1)
    def _():
        pltpu.async_copy(zero_hbm.at[pl.ds(0, TAIL)],
                         acc.at[pl.ds(NS * RPT, TAIL)], sem_z)

    def fire_idx(i, ref, sem):
        base = pl.multiple_of(eoff + i * BB, 8)
        pltpu.async_copy(row_hbm.at[pl.ds(base, BB)], ref.at[0], sem)
        pltpu.async_copy(col_hbm.at[pl.ds(base, BB)], ref.at[1], sem)

    def drain_idx(i, ref, sem):
        base = pl.multiple_of(eoff + i * BB, 8)
        pltpu.make_async_copy(row_hbm.at[pl.ds(base, BB)], ref.at[0],
                              sem).wait()
        pltpu.make_async_copy(col_hbm.at[pl.ds(base, BB)], ref.at[1],
                              sem).wait()

    fire_idx(0, idxA, sem_ia)
    fire_idx(1, idxB, sem_ib)
    drain_idx(0, idxA, sem_ia)
    dz.wait()

    @pl.when(s == NS - 1)
    def _():
        pltpu.make_async_copy(zero_hbm.at[pl.ds(0, TAIL)],
                              acc.at[pl.ds(NS * RPT, TAIL)], sem_z).wait()

    plsc.subcore_barrier()

    def body(t, carry):
        a = 2 * t
        b = a + 1
        last = t + 1 >= TPAIR
        pltpu.async_copy(hp_hbm.at[idxA.at[0]], rowsA, sem_ga)
        drain_idx(b, idxB, sem_ib)
        pltpu.make_async_copy(hp_hbm.at[idxA.at[0]], rowsA, sem_ga).wait()
        pltpu.async_copy(rowsA, acc.at[idxA.at[1]], sem_sa, add=True)
        pltpu.async_copy(hp_hbm.at[idxB.at[0]], rowsB, sem_gb)
        pltpu.make_async_copy(rowsA, acc.at[idxA.at[1]], sem_sa).wait()

        @pl.when(jnp.logical_not(last))
        def _():
            fire_idx(a + 2, idxA, sem_ia)

        pltpu.make_async_copy(hp_hbm.at[idxB.at[0]], rowsB, sem_gb).wait()
        pltpu.async_copy(rowsB, acc.at[idxB.at[1]], sem_sb, add=True)
        pltpu.make_async_copy(rowsB, acc.at[idxB.at[1]], sem_sb).wait()

        @pl.when(jnp.logical_not(last))
        def _():
            fire_idx(b + 2, idxB, sem_ib)
            drain_idx(a + 2, idxA, sem_ia)

        return carry

    lax.fori_loop(0, TPAIR, body, 0)

    @pl.when(w < WX)
    def _():
        base = pl.multiple_of(XOFF + w * BB, 8)
        pltpu.sync_copy(row_hbm.at[pl.ds(base, BB)], idxA.at[0])
        pltpu.sync_copy(col_hbm.at[pl.ds(base, BB)], idxA.at[1])
        pltpu.sync_copy(hp_hbm.at[idxA.at[0]], rowsA)
        pltpu.sync_copy(rowsA, acc.at[idxA.at[1]], add=True)

    plsc.subcore_barrier()
    pltpu.sync_copy(acc.at[pl.ds(rbase, RPT)],
                    out_hbm.at[c, pl.ds(rbase, RPT)])

    @pl.when(s == NS - 1)
    def _():
        pltpu.sync_copy(acc.at[pl.ds(NS * RPT, TAIL)],
                        out_hbm.at[c, pl.ds(NS * RPT, TAIL)])


def _first_body(cnt_ref, x_ref, w_ref, h_ref, hp_ref, dinv_ref):
    deg = jnp.sum(cnt_ref[0] + cnt_ref[1], axis=1, keepdims=True) + 2.0
    dinv = lax.rsqrt(deg)
    h = jnp.dot(x_ref[...], w_ref[...], preferred_element_type=jnp.float32)
    h_ref[...] = h
    hp_ref[...] = h * dinv
    dinv_ref[...] = dinv


def _tc_first(cnt, x, W1):
    return pl.pallas_call(
        _first_body,
        out_shape=[
            jax.ShapeDtypeStruct((N, D), jnp.float32),
            jax.ShapeDtypeStruct((N, D), jnp.float32),
            jax.ShapeDtypeStruct((N, 1), jnp.float32),
        ],
    )(cnt, x, W1)


def _mid_body(agg_ref, hprev_ref, dinv_ref, b_ref, w_ref, h_ref, hp_ref):
    dinv = dinv_ref[...]
    z = (dinv * (agg_ref[0] + agg_ref[1])
         + (2.0 * dinv * dinv) * hprev_ref[...] + b_ref[...])
    h = jnp.dot(z, w_ref[...], preferred_element_type=jnp.float32)
    h_ref[...] = h
    hp_ref[...] = h * dinv


def _tc_mid(agg, hprev, dinv, b2d, W):
    return pl.pallas_call(
        _mid_body,
        out_shape=[
            jax.ShapeDtypeStruct((N, D), jnp.float32),
            jax.ShapeDtypeStruct((N, D), jnp.float32),
        ],
    )(agg, hprev, dinv, b2d, W)


def _final_body(agg_ref, hprev_ref, dinv_ref, b_ref, out_ref):
    dinv = dinv_ref[...]
    out_ref[...] = (dinv * (agg_ref[0] + agg_ref[1])
                    + (2.0 * dinv * dinv) * hprev_ref[...] + b_ref[...])


def _tc_final(agg, hprev, dinv, b2d):
    return pl.pallas_call(
        _final_body,
        out_shape=jax.ShapeDtypeStruct((N, D), jnp.float32),
    )(agg, hprev, dinv, b2d)


def kernel(x, edge_index, W1, b1, W2, b2):
    row = edge_index[0]
    col = edge_index[1]
    zeros = jnp.zeros((RPT, D), jnp.float32)
    b1d = b1.reshape(1, D)
    b2d = b2.reshape(1, D)

    cnt = _sc_hist(col, zeros)
    h, hp, dinv = _tc_first(cnt, x, W1)
    for b in (b1d, b2d, b2d, b2d):
        agg = _sc_agg(hp, row, col, zeros)
        h, hp = _tc_mid(agg, h, dinv, b, W2)
    agg = _sc_agg(hp, row, col, zeros)
    return _tc_final(agg, h, dinv, b2d)


# speedup vs baseline: 19.8014x; 1.1337x over previous
import functools

import jax
import jax.numpy as jnp
from jax import lax
from jax.experimental import pallas as pl
from jax.experimental.pallas import tpu as pltpu
from jax.experimental.pallas import tpu_sc as plsc

N = 10000
E = 320000
D = 128

NC = 2
NS = 16
NW = NC * NS
BB = 128
GB = E // BB
WB = GB // NW
WX = GB - WB * NW
TPAIR = WB // 2
SLAB = WB * BB
XOFF = NW * SLAB
RPT = 624
TAIL = N - NS * RPT

_mesh = plsc.VectorSubcoreMesh(core_axis_name="c", subcore_axis_name="s")


@functools.partial(
    pl.kernel,
    out_type=jax.ShapeDtypeStruct((NC, N, D), jnp.float32),
    mesh=_mesh,
    scratch_types=[
        pltpu.VMEM((WB + 1, BB), jnp.int32),
        pltpu.VMEM((BB, D), jnp.float32),
        pltpu.VMEM_SHARED((N, D), jnp.float32),
        pltpu.SemaphoreType.DMA,
        pltpu.SemaphoreType.DMA,
        pltpu.SemaphoreType.DMA,
    ],
)
def _sc_hist(col_hbm, zero_hbm, out_hbm, cidx, ones, acc, sem_z, sem_i,
             sem_s):
    c = lax.axis_index("c")
    s = lax.axis_index("s")
    w = s * NC + c
    xtra = w < WX
    nb = WB + jnp.where(xtra, 1, 0)
    rbase = pl.multiple_of(s * RPT, 8)
    eoff = w * SLAB

    dz = pltpu.async_copy(zero_hbm, acc.at[pl.ds(rbase, RPT)], sem_z)

    @pl.when(s == NS - 1)
    def _():
        pltpu.async_copy(zero_hbm.at[pl.ds(0, TAIL)],
                         acc.at[pl.ds(NS * RPT, TAIL)], sem_z)

    def iload(i, carry):
        pltpu.async_copy(col_hbm.at[pl.ds(eoff + i * BB, BB)], cidx.at[i],
                         sem_i)
        return carry

    lax.fori_loop(0, WB, iload, 0)

    @pl.when(xtra)
    def _():
        pltpu.async_copy(col_hbm.at[pl.ds(XOFF + w * BB, BB)], cidx.at[WB],
                         sem_i)

    lane = lax.iota(jnp.int32, 16)
    onehot = jnp.where(lane == 0, 1.0, 0.0).astype(jnp.float32)
    zero = jnp.zeros((16,), jnp.float32)

    def init(i, carry):
        ones[i, pl.ds(0, 16)] = onehot
        for j in range(1, D // 16):
            ones[i, pl.ds(j * 16, 16)] = zero
        return carry

    lax.fori_loop(0, BB, init, 0)

    def idrain(i, carry):
        pltpu.make_async_copy(col_hbm.at[pl.ds(eoff, BB)], cidx.at[0],
                              sem_i).wait()
        return carry

    lax.fori_loop(0, nb, idrain, 0)
    dz.wait()

    @pl.when(s == NS - 1)
    def _():
        pltpu.make_async_copy(zero_hbm.at[pl.ds(0, TAIL)],
                              acc.at[pl.ds(NS * RPT, TAIL)], sem_z).wait()

    plsc.subcore_barrier()

    def body(i, carry):
        pltpu.async_copy(ones, acc.at[cidx.at[i]], sem_s, add=True)

        @pl.when(i >= 2)
        def _():
            pltpu.make_async_copy(ones, acc.at[cidx.at[i - 2]], sem_s).wait()

        return carry

    lax.fori_loop(0, nb, body, 0)

    def drain(i, carry):
        pltpu.make_async_copy(ones, acc.at[cidx.at[nb - 2 + i]], sem_s).wait()
        return carry

    lax.fori_loop(0, 2, drain, 0)
    plsc.subcore_barrier()
    pltpu.sync_copy(acc.at[pl.ds(rbase, RPT)],
                    out_hbm.at[c, pl.ds(rbase, RPT)])

    @pl.when(s == NS - 1)
    def _():
        pltpu.sync_copy(acc.at[pl.ds(NS * RPT, TAIL)],
                        out_hbm.at[c, pl.ds(NS * RPT, TAIL)])


@functools.partial(
    pl.kernel,
    out_type=jax.ShapeDtypeStruct((NC, N, D), jnp.float32),
    mesh=_mesh,
    scratch_types=[
        pltpu.VMEM((6, 2, BB), jnp.int32),
        pltpu.VMEM((2, BB, D), jnp.float32),
        pltpu.VMEM_SHARED((N, D), jnp.float32),
        pltpu.SemaphoreType.DMA,
        pltpu.SemaphoreType.DMA,
        pltpu.SemaphoreType.DMA,
        pltpu.SemaphoreType.DMA,
        pltpu.SemaphoreType.DMA,
        pltpu.SemaphoreType.DMA,
        pltpu.SemaphoreType.DMA,
        pltpu.SemaphoreType.DMA,
        pltpu.SemaphoreType.DMA,
        pltpu.SemaphoreType.DMA,
        pltpu.SemaphoreType.DMA,
    ],
)
def _sc_agg(hp_hbm, row_hbm, col_hbm, zero_hbm, out_hbm,
            idx, rows, acc, sem_z, si0, si1, si2, si3, si4, si5,
            sg0, sg1, ss0, ss1):
    c = lax.axis_index("c")
    s = lax.axis_index("s")
    w = s * NC + c
    rbase = pl.multiple_of(s * RPT, 8)
    eoff = w * SLAB
    sem_i = (si0, si1, si2, si3, si4, si5)
    sem_g = (sg0, sg1)
    sem_s = (ss0, ss1)

    dz = pltpu.async_copy(zero_hbm, acc.at[pl.ds(rbase, RPT)], sem_z)

    @pl.when(s == NS - 1)
    def _():
        pltpu.async_copy(zero_hbm.at[pl.ds(0, TAIL)],
                         acc.at[pl.ds(NS * RPT, TAIL)], sem_z)

    def fire_idx(i, slot):
        base = pl.multiple_of(eoff + i * BB, 8)
        pltpu.async_copy(row_hbm.at[pl.ds(base, BB)], idx.at[slot, 0],
                         sem_i[slot])
        pltpu.async_copy(col_hbm.at[pl.ds(base, BB)], idx.at[slot, 1],
                         sem_i[slot])

    def drain_idx(i, slot):
        base = pl.multiple_of(eoff + i * BB, 8)
        pltpu.make_async_copy(row_hbm.at[pl.ds(base, BB)], idx.at[slot, 0],
                              sem_i[slot]).wait()
        pltpu.make_async_copy(col_hbm.at[pl.ds(base, BB)], idx.at[slot, 1],
                              sem_i[slot]).wait()

    def fire_gather(slot):
        pltpu.async_copy(hp_hbm.at[idx.at[slot, 0]], rows.at[slot % 2],
                         sem_g[slot % 2])

    def drain_gather(slot):
        pltpu.make_async_copy(hp_hbm.at[idx.at[slot, 0]], rows.at[slot % 2],
                              sem_g[slot % 2]).wait()

    def fire_scatter(slot):
        pltpu.async_copy(rows.at[slot % 2], acc.at[idx.at[slot, 1]],
                         sem_s[slot % 2], add=True)

    def drain_scatter(slot):
        pltpu.make_async_copy(rows.at[slot % 2], acc.at[idx.at[slot, 1]],
                              sem_s[slot % 2]).wait()

    for k in range(5):
        fire_idx(k, k)
    drain_idx(0, 0)
    fire_gather(0)
    dz.wait()

    @pl.when(s == NS - 1)
    def _():
        pltpu.make_async_copy(zero_hbm.at[pl.ds(0, TAIL)],
                              acc.at[pl.ds(NS * RPT, TAIL)], sem_z).wait()

    plsc.subcore_barrier()

    def body(t, carry):
        for k in range(6):
            i = 6 * t + k
            drain_gather(k)
            fire_scatter(k)

            @pl.when(i > 0)
            def _():
                drain_scatter((k + 5) % 6)

            @pl.when(i + 1 < WB)
            def _():
                drain_idx(i + 1, (k + 1) % 6)
                fire_gather((k + 1) % 6)

            @pl.when(i + 5 < WB)
            def _():
                fire_idx(i + 5, (k + 5) % 6)

        return carry

    lax.fori_loop(0, WB // 6, body, 0)
    drain_scatter((WB - 1) % 6)

    @pl.when(w < WX)
    def _():
        base = pl.multiple_of(XOFF + w * BB, 8)
        pltpu.sync_copy(row_hbm.at[pl.ds(base, BB)], idx.at[0, 0])
        pltpu.sync_copy(col_hbm.at[pl.ds(base, BB)], idx.at[0, 1])
        pltpu.sync_copy(hp_hbm.at[idx.at[0, 0]], rows.at[0])
        pltpu.sync_copy(rows.at[0], acc.at[idx.at[0, 1]], add=True)

    plsc.subcore_barrier()
    pltpu.sync_copy(acc.at[pl.ds(rbase, RPT)],
                    out_hbm.at[c, pl.ds(rbase, RPT)])

    @pl.when(s == NS - 1)
    def _():
        pltpu.sync_copy(acc.at[pl.ds(NS * RPT, TAIL)],
                        out_hbm.at[c, pl.ds(NS * RPT, TAIL)])


def _first_body(cnt_ref, x_ref, w_ref, h_ref, hp_ref, dinv_ref):
    deg = jnp.sum(cnt_ref[0] + cnt_ref[1], axis=1, keepdims=True) + 2.0
    dinv = lax.rsqrt(deg)
    h = jnp.dot(x_ref[...], w_ref[...], preferred_element_type=jnp.float32)
    h_ref[...] = h
    hp_ref[...] = h * dinv
    dinv_ref[...] = dinv


def _tc_first(cnt, x, W1):
    return pl.pallas_call(
        _first_body,
        out_shape=[
            jax.ShapeDtypeStruct((N, D), jnp.float32),
            jax.ShapeDtypeStruct((N, D), jnp.float32),
            jax.ShapeDtypeStruct((N, 1), jnp.float32),
        ],
    )(cnt, x, W1)


def _mid_body(agg_ref, hprev_ref, dinv_ref, b_ref, w_ref, h_ref, hp_ref):
    dinv = dinv_ref[...]
    z = (dinv * (agg_ref[0] + agg_ref[1])
         + (2.0 * dinv * dinv) * hprev_ref[...] + b_ref[...])
    h = jnp.dot(z, w_ref[...], preferred_element_type=jnp.float32)
    h_ref[...] = h
    hp_ref[...] = h * dinv


def _tc_mid(agg, hprev, dinv, b2d, W):
    return pl.pallas_call(
        _mid_body,
        out_shape=[
            jax.ShapeDtypeStruct((N, D), jnp.float32),
            jax.ShapeDtypeStruct((N, D), jnp.float32),
        ],
    )(agg, hprev, dinv, b2d, W)


def _final_body(agg_ref, hprev_ref, dinv_ref, b_ref, out_ref):
    dinv = dinv_ref[...]
    out_ref[...] = (dinv * (agg_ref[0] + agg_ref[1])
                    + (2.0 * dinv * dinv) * hprev_ref[...] + b_ref[...])


def _tc_final(agg, hprev, dinv, b2d):
    return pl.pallas_call(
        _final_body,
        out_shape=jax.ShapeDtypeStruct((N, D), jnp.float32),
    )(agg, hprev, dinv, b2d)


def kernel(x, edge_index, W1, b1, W2, b2):
    row = edge_index[0]
    col = edge_index[1]
    zeros = jnp.zeros((RPT, D), jnp.float32)
    b1d = b1.reshape(1, D)
    b2d = b2.reshape(1, D)

    cnt = _sc_hist(col, zeros)
    h, hp, dinv = _tc_first(cnt, x, W1)
    for b in (b1d, b2d, b2d, b2d):
        agg = _sc_agg(hp, row, col, zeros)
        h, hp = _tc_mid(agg, h, dinv, b, W2)
    agg = _sc_agg(hp, row, col, zeros)
    return _tc_final(agg, h, dinv, b2d)


# speedup vs baseline: 19.9753x; 1.0088x over previous
import functools

import jax
import jax.numpy as jnp
from jax import lax
from jax.experimental import pallas as pl
from jax.experimental.pallas import tpu as pltpu
from jax.experimental.pallas import tpu_sc as plsc

N = 10000
E = 320000
D = 128

NC = 2
NS = 16
NW = NC * NS
BB = 128
GB = E // BB
WB = GB // NW
WX = GB - WB * NW
TPAIR = WB // 2
SLAB = WB * BB
XOFF = NW * SLAB
RPT = 624
TAIL = N - NS * RPT

_mesh = plsc.VectorSubcoreMesh(core_axis_name="c", subcore_axis_name="s")


@functools.partial(
    pl.kernel,
    out_type=jax.ShapeDtypeStruct((NC, N, D), jnp.float32),
    mesh=_mesh,
    scratch_types=[
        pltpu.VMEM((WB + 1, BB), jnp.int32),
        pltpu.VMEM((BB, D), jnp.float32),
        pltpu.VMEM_SHARED((N, D), jnp.float32),
        pltpu.SemaphoreType.DMA,
        pltpu.SemaphoreType.DMA,
        pltpu.SemaphoreType.DMA,
    ],
)
def _sc_hist(col_hbm, zero_hbm, out_hbm, cidx, ones, acc, sem_z, sem_i,
             sem_s):
    c = lax.axis_index("c")
    s = lax.axis_index("s")
    w = s * NC + c
    xtra = w < WX
    nb = WB + jnp.where(xtra, 1, 0)
    rbase = pl.multiple_of(s * RPT, 8)
    eoff = w * SLAB

    dz = pltpu.async_copy(zero_hbm, acc.at[pl.ds(rbase, RPT)], sem_z)

    @pl.when(s == NS - 1)
    def _():
        pltpu.async_copy(zero_hbm.at[pl.ds(0, TAIL)],
                         acc.at[pl.ds(NS * RPT, TAIL)], sem_z)

    def iload(i, carry):
        pltpu.async_copy(col_hbm.at[pl.ds(eoff + i * BB, BB)], cidx.at[i],
                         sem_i)
        return carry

    lax.fori_loop(0, WB, iload, 0)

    @pl.when(xtra)
    def _():
        pltpu.async_copy(col_hbm.at[pl.ds(XOFF + w * BB, BB)], cidx.at[WB],
                         sem_i)

    lane = lax.iota(jnp.int32, 16)
    onehot = jnp.where(lane == 0, 1.0, 0.0).astype(jnp.float32)
    zero = jnp.zeros((16,), jnp.float32)

    def init(i, carry):
        ones[i, pl.ds(0, 16)] = onehot
        for j in range(1, D // 16):
            ones[i, pl.ds(j * 16, 16)] = zero
        return carry

    lax.fori_loop(0, BB, init, 0)

    def idrain(i, carry):
        pltpu.make_async_copy(col_hbm.at[pl.ds(eoff, BB)], cidx.at[0],
                              sem_i).wait()
        return carry

    lax.fori_loop(0, nb, idrain, 0)
    dz.wait()

    @pl.when(s == NS - 1)
    def _():
        pltpu.make_async_copy(zero_hbm.at[pl.ds(0, TAIL)],
                              acc.at[pl.ds(NS * RPT, TAIL)], sem_z).wait()

    plsc.subcore_barrier()

    def body(i, carry):
        pltpu.async_copy(ones, acc.at[cidx.at[i]], sem_s, add=True)

        @pl.when(i >= 2)
        def _():
            pltpu.make_async_copy(ones, acc.at[cidx.at[i - 2]], sem_s).wait()

        return carry

    lax.fori_loop(0, nb, body, 0)

    def drain(i, carry):
        pltpu.make_async_copy(ones, acc.at[cidx.at[nb - 2 + i]], sem_s).wait()
        return carry

    lax.fori_loop(0, 2, drain, 0)
    plsc.subcore_barrier()
    pltpu.sync_copy(acc.at[pl.ds(rbase, RPT)],
                    out_hbm.at[c, pl.ds(rbase, RPT)])

    @pl.when(s == NS - 1)
    def _():
        pltpu.sync_copy(acc.at[pl.ds(NS * RPT, TAIL)],
                        out_hbm.at[c, pl.ds(NS * RPT, TAIL)])


@functools.partial(
    pl.kernel,
    out_type=jax.ShapeDtypeStruct((NC, N, D), jnp.float32),
    mesh=_mesh,
    scratch_types=[
        pltpu.VMEM((6, 2, BB), jnp.int32),
        pltpu.VMEM((2, BB, D), jnp.float32),
        pltpu.VMEM_SHARED((N, D), jnp.float32),
        pltpu.SemaphoreType.DMA,
        pltpu.SemaphoreType.DMA,
        pltpu.SemaphoreType.DMA,
        pltpu.SemaphoreType.DMA,
        pltpu.SemaphoreType.DMA,
        pltpu.SemaphoreType.DMA,
        pltpu.SemaphoreType.DMA,
        pltpu.SemaphoreType.DMA,
        pltpu.SemaphoreType.DMA,
        pltpu.SemaphoreType.DMA,
        pltpu.SemaphoreType.DMA,
    ],
)
def _sc_agg(hp_hbm, eidx_hbm, zero_hbm, out_hbm,
            idx, rows, acc, sem_z, si0, si1, si2, si3, si4, si5,
            sg0, sg1, ss0, ss1):
    c = lax.axis_index("c")
    s = lax.axis_index("s")
    w = s * NC + c
    rbase = pl.multiple_of(s * RPT, 8)
    eoff = w * SLAB
    sem_i = (si0, si1, si2, si3, si4, si5)
    sem_g = (sg0, sg1)
    sem_s = (ss0, ss1)

    dz = pltpu.async_copy(zero_hbm, acc.at[pl.ds(rbase, RPT)], sem_z)

    @pl.when(s == NS - 1)
    def _():
        pltpu.async_copy(zero_hbm.at[pl.ds(0, TAIL)],
                         acc.at[pl.ds(NS * RPT, TAIL)], sem_z)

    def fire_idx(i, slot):
        base = pl.multiple_of(eoff + i * BB, BB)
        pltpu.async_copy(eidx_hbm.at[:, pl.ds(base, BB)], idx.at[slot],
                         sem_i[slot])

    def drain_idx(i, slot):
        base = pl.multiple_of(eoff + i * BB, BB)
        pltpu.make_async_copy(eidx_hbm.at[:, pl.ds(base, BB)], idx.at[slot],
                              sem_i[slot]).wait()

    def fire_gather(slot):
        pltpu.async_copy(hp_hbm.at[idx.at[slot, 0]], rows.at[slot % 2],
                         sem_g[slot % 2])

    def drain_gather(slot):
        pltpu.make_async_copy(hp_hbm.at[idx.at[slot, 0]], rows.at[slot % 2],
                              sem_g[slot % 2]).wait()

    def fire_scatter(slot):
        pltpu.async_copy(rows.at[slot % 2], acc.at[idx.at[slot, 1]],
                         sem_s[slot % 2], add=True)

    def drain_scatter(slot):
        pltpu.make_async_copy(rows.at[slot % 2], acc.at[idx.at[slot, 1]],
                              sem_s[slot % 2]).wait()

    for k in range(5):
        fire_idx(k, k)
    drain_idx(0, 0)
    fire_gather(0)
    dz.wait()

    @pl.when(s == NS - 1)
    def _():
        pltpu.make_async_copy(zero_hbm.at[pl.ds(0, TAIL)],
                              acc.at[pl.ds(NS * RPT, TAIL)], sem_z).wait()

    plsc.subcore_barrier()

    def body(t, carry):
        for k in range(6):
            i = 6 * t + k
            drain_gather(k)
            fire_scatter(k)

            @pl.when(i > 0)
            def _():
                drain_scatter((k + 5) % 6)

            @pl.when(i + 1 < WB)
            def _():
                drain_idx(i + 1, (k + 1) % 6)
                fire_gather((k + 1) % 6)

            @pl.when(i + 5 < WB)
            def _():
                fire_idx(i + 5, (k + 5) % 6)

        return carry

    lax.fori_loop(0, WB // 6, body, 0)
    drain_scatter((WB - 1) % 6)

    @pl.when(w < WX)
    def _():
        base = pl.multiple_of(XOFF + w * BB, BB)
        pltpu.sync_copy(eidx_hbm.at[:, pl.ds(base, BB)], idx.at[0])
        pltpu.sync_copy(hp_hbm.at[idx.at[0, 0]], rows.at[0])
        pltpu.sync_copy(rows.at[0], acc.at[idx.at[0, 1]], add=True)

    plsc.subcore_barrier()
    pltpu.sync_copy(acc.at[pl.ds(rbase, RPT)],
                    out_hbm.at[c, pl.ds(rbase, RPT)])

    @pl.when(s == NS - 1)
    def _():
        pltpu.sync_copy(acc.at[pl.ds(NS * RPT, TAIL)],
                        out_hbm.at[c, pl.ds(NS * RPT, TAIL)])


def _mm1_body(x_ref, w_ref, h_ref):
    h_ref[...] = jnp.dot(x_ref[...], w_ref[...],
                         preferred_element_type=jnp.float32)


def _tc_mm1(x, W1):
    return pl.pallas_call(
        _mm1_body,
        out_shape=jax.ShapeDtypeStruct((N, D), jnp.float32),
    )(x, W1)


def _scale_body(cnt_ref, h_ref, hp_ref, dinv_ref):
    deg = jnp.sum(cnt_ref[0] + cnt_ref[1], axis=1, keepdims=True) + 2.0
    dinv = lax.rsqrt(deg)
    hp_ref[...] = h_ref[...] * dinv
    dinv_ref[...] = dinv


def _tc_scale(cnt, h):
    return pl.pallas_call(
        _scale_body,
        out_shape=[
            jax.ShapeDtypeStruct((N, D), jnp.float32),
            jax.ShapeDtypeStruct((N, 1), jnp.float32),
        ],
    )(cnt, h)


def _mid_body(agg_ref, hprev_ref, dinv_ref, b_ref, w_ref, h_ref, hp_ref):
    dinv = dinv_ref[...]
    z = (dinv * (agg_ref[0] + agg_ref[1])
         + (2.0 * dinv * dinv) * hprev_ref[...] + b_ref[...])
    h = jnp.dot(z, w_ref[...], preferred_element_type=jnp.float32)
    h_ref[...] = h
    hp_ref[...] = h * dinv


def _tc_mid(agg, hprev, dinv, b2d, W):
    return pl.pallas_call(
        _mid_body,
        out_shape=[
            jax.ShapeDtypeStruct((N, D), jnp.float32),
            jax.ShapeDtypeStruct((N, D), jnp.float32),
        ],
    )(agg, hprev, dinv, b2d, W)


def _final_body(agg_ref, hprev_ref, dinv_ref, b_ref, out_ref):
    dinv = dinv_ref[...]
    out_ref[...] = (dinv * (agg_ref[0] + agg_ref[1])
                    + (2.0 * dinv * dinv) * hprev_ref[...] + b_ref[...])


def _tc_final(agg, hprev, dinv, b2d):
    return pl.pallas_call(
        _final_body,
        out_shape=jax.ShapeDtypeStruct((N, D), jnp.float32),
    )(agg, hprev, dinv, b2d)


def kernel(x, edge_index, W1, b1, W2, b2):
    col = edge_index[1]
    zeros = jnp.zeros((RPT, D), jnp.float32)
    b1d = b1.reshape(1, D)
    b2d = b2.reshape(1, D)

    h = _tc_mm1(x, W1)
    cnt = _sc_hist(col, zeros)
    hp, dinv = _tc_scale(cnt, h)
    for b in (b1d, b2d, b2d, b2d):
        agg = _sc_agg(hp, edge_index, zeros)
        h, hp = _tc_mid(agg, h, dinv, b, W2)
    agg = _sc_agg(hp, edge_index, zeros)
    return _tc_final(agg, h, dinv, b2d)


# speedup vs baseline: 23.1259x; 1.1577x over previous
import functools

import jax
import jax.numpy as jnp
from jax import lax
from jax.experimental import pallas as pl
from jax.experimental.pallas import tpu as pltpu
from jax.experimental.pallas import tpu_sc as plsc

N = 10000
E = 320000
D = 128

NC = 2
NS = 16
NW = NC * NS
BB = 128
GB = E // BB
WB = GB // NW
WX = GB - WB * NW
TPAIR = WB // 2
SLAB = WB * BB
XOFF = NW * SLAB
RPT = 624
TAIL = N - NS * RPT

_mesh = plsc.VectorSubcoreMesh(core_axis_name="c", subcore_axis_name="s")


@functools.partial(
    pl.kernel,
    out_type=jax.ShapeDtypeStruct((NC, N, D), jnp.float32),
    mesh=_mesh,
    scratch_types=[
        pltpu.VMEM((WB + 1, BB), jnp.int32),
        pltpu.VMEM((BB, D), jnp.float32),
        pltpu.VMEM_SHARED((N, D), jnp.float32),
        pltpu.SemaphoreType.DMA,
        pltpu.SemaphoreType.DMA,
        pltpu.SemaphoreType.DMA,
    ],
)
def _sc_hist(col_hbm, zero_hbm, out_hbm, cidx, ones, acc, sem_z, sem_i,
             sem_s):
    c = lax.axis_index("c")
    s = lax.axis_index("s")
    w = s * NC + c
    xtra = w < WX
    nb = WB + jnp.where(xtra, 1, 0)
    rbase = pl.multiple_of(s * RPT, 8)
    eoff = w * SLAB

    dz = pltpu.async_copy(zero_hbm, acc.at[pl.ds(rbase, RPT)], sem_z)

    @pl.when(s == NS - 1)
    def _():
        pltpu.async_copy(zero_hbm.at[pl.ds(0, TAIL)],
                         acc.at[pl.ds(NS * RPT, TAIL)], sem_z)

    def iload(i, carry):
        pltpu.async_copy(col_hbm.at[pl.ds(eoff + i * BB, BB)], cidx.at[i],
                         sem_i)
        return carry

    lax.fori_loop(0, WB, iload, 0)

    @pl.when(xtra)
    def _():
        pltpu.async_copy(col_hbm.at[pl.ds(XOFF + w * BB, BB)], cidx.at[WB],
                         sem_i)

    lane = lax.iota(jnp.int32, 16)
    onehot = jnp.where(lane == 0, 1.0, 0.0).astype(jnp.float32)
    zero = jnp.zeros((16,), jnp.float32)

    def init(i, carry):
        ones[i, pl.ds(0, 16)] = onehot
        for j in range(1, D // 16):
            ones[i, pl.ds(j * 16, 16)] = zero
        return carry

    lax.fori_loop(0, BB, init, 0)

    def idrain(i, carry):
        pltpu.make_async_copy(col_hbm.at[pl.ds(eoff, BB)], cidx.at[0],
                              sem_i).wait()
        return carry

    lax.fori_loop(0, nb, idrain, 0)
    dz.wait()

    @pl.when(s == NS - 1)
    def _():
        pltpu.make_async_copy(zero_hbm.at[pl.ds(0, TAIL)],
                              acc.at[pl.ds(NS * RPT, TAIL)], sem_z).wait()

    plsc.subcore_barrier()

    def body(i, carry):
        pltpu.async_copy(ones, acc.at[cidx.at[i]], sem_s, add=True)

        @pl.when(i >= 2)
        def _():
            pltpu.make_async_copy(ones, acc.at[cidx.at[i - 2]], sem_s).wait()

        return carry

    lax.fori_loop(0, nb, body, 0)

    def drain(i, carry):
        pltpu.make_async_copy(ones, acc.at[cidx.at[nb - 2 + i]], sem_s).wait()
        return carry

    lax.fori_loop(0, 2, drain, 0)
    plsc.subcore_barrier()
    pltpu.sync_copy(acc.at[pl.ds(rbase, RPT)],
                    out_hbm.at[c, pl.ds(rbase, RPT)])

    @pl.when(s == NS - 1)
    def _():
        pltpu.sync_copy(acc.at[pl.ds(NS * RPT, TAIL)],
                        out_hbm.at[c, pl.ds(NS * RPT, TAIL)])


@functools.partial(
    pl.kernel,
    out_type=jax.ShapeDtypeStruct((NC, N, D), jnp.float32),
    mesh=_mesh,
    scratch_types=[
        pltpu.VMEM((6, 2, BB), jnp.int32),
        pltpu.VMEM((2, BB, D), jnp.float32),
        pltpu.VMEM_SHARED((N, D), jnp.float32),
        pltpu.SemaphoreType.DMA,
        pltpu.SemaphoreType.DMA,
        pltpu.SemaphoreType.DMA,
        pltpu.SemaphoreType.DMA,
        pltpu.SemaphoreType.DMA,
        pltpu.SemaphoreType.DMA,
        pltpu.SemaphoreType.DMA,
        pltpu.SemaphoreType.DMA,
        pltpu.SemaphoreType.DMA,
        pltpu.SemaphoreType.DMA,
        pltpu.SemaphoreType.DMA,
    ],
)
def _sc_agg(hp_hbm, eidx_hbm, zero_hbm, out_hbm,
            idx, rows, acc, sem_z, si0, si1, si2, si3, si4, si5,
            sg0, sg1, ss0, ss1):
    c = lax.axis_index("c")
    s = lax.axis_index("s")
    w = s * NC + c
    rbase = pl.multiple_of(s * RPT, 8)
    eoff = w * SLAB
    sem_i = (si0, si1, si2, si3, si4, si5)
    sem_g = (sg0, sg1)
    sem_s = (ss0, ss1)

    dz = pltpu.async_copy(zero_hbm, acc.at[pl.ds(rbase, RPT)], sem_z)

    @pl.when(s == NS - 1)
    def _():
        pltpu.async_copy(zero_hbm.at[pl.ds(0, TAIL)],
                         acc.at[pl.ds(NS * RPT, TAIL)], sem_z)

    def fire_idx(i, slot):
        base = pl.multiple_of(eoff + i * BB, BB)
        pltpu.async_copy(eidx_hbm.at[:, pl.ds(base, BB)], idx.at[slot],
                         sem_i[slot])

    def drain_idx(i, slot):
        base = pl.multiple_of(eoff + i * BB, BB)
        pltpu.make_async_copy(eidx_hbm.at[:, pl.ds(base, BB)], idx.at[slot],
                              sem_i[slot]).wait()

    def fire_gather(slot):
        pltpu.async_copy(hp_hbm.at[idx.at[slot, 0]], rows.at[slot % 2],
                         sem_g[slot % 2])

    def drain_gather(slot):
        pltpu.make_async_copy(hp_hbm.at[idx.at[slot, 0]], rows.at[slot % 2],
                              sem_g[slot % 2]).wait()

    def fire_scatter(slot):
        pltpu.async_copy(rows.at[slot % 2], acc.at[idx.at[slot, 1]],
                         sem_s[slot % 2], add=True)

    def drain_scatter(slot):
        pltpu.make_async_copy(rows.at[slot % 2], acc.at[idx.at[slot, 1]],
                              sem_s[slot % 2]).wait()

    for k in range(5):
        fire_idx(k, k)
    drain_idx(0, 0)
    fire_gather(0)
    dz.wait()

    @pl.when(s == NS - 1)
    def _():
        pltpu.make_async_copy(zero_hbm.at[pl.ds(0, TAIL)],
                              acc.at[pl.ds(NS * RPT, TAIL)], sem_z).wait()

    plsc.subcore_barrier()

    def body(t, carry):
        for k in range(6):
            i = 6 * t + k

            @pl.when(i > 0)
            def _():
                drain_scatter((k + 5) % 6)

            @pl.when(i + 1 < WB)
            def _():
                drain_idx(i + 1, (k + 1) % 6)
                fire_gather((k + 1) % 6)

            drain_gather(k)
            fire_scatter(k)

            @pl.when(i + 5 < WB)
            def _():
                fire_idx(i + 5, (k + 5) % 6)

        return carry

    lax.fori_loop(0, WB // 6, body, 0)
    drain_scatter((WB - 1) % 6)

    @pl.when(w < WX)
    def _():
        base = pl.multiple_of(XOFF + w * BB, BB)
        pltpu.sync_copy(eidx_hbm.at[:, pl.ds(base, BB)], idx.at[0])
        pltpu.sync_copy(hp_hbm.at[idx.at[0, 0]], rows.at[0])
        pltpu.sync_copy(rows.at[0], acc.at[idx.at[0, 1]], add=True)

    plsc.subcore_barrier()
    pltpu.sync_copy(acc.at[pl.ds(rbase, RPT)],
                    out_hbm.at[c, pl.ds(rbase, RPT)])

    @pl.when(s == NS - 1)
    def _():
        pltpu.sync_copy(acc.at[pl.ds(NS * RPT, TAIL)],
                        out_hbm.at[c, pl.ds(NS * RPT, TAIL)])


def _mm1_body(x_ref, w_ref, h_ref):
    h_ref[...] = jnp.dot(x_ref[...], w_ref[...],
                         preferred_element_type=jnp.float32)


def _tc_mm1(x, W1):
    return pl.pallas_call(
        _mm1_body,
        out_shape=jax.ShapeDtypeStruct((N, D), jnp.float32),
    )(x, W1)


def _scale_body(cnt_ref, h_ref, hp_ref, dinv_ref):
    deg = jnp.sum(cnt_ref[0] + cnt_ref[1], axis=1, keepdims=True) + 2.0
    dinv = lax.rsqrt(deg)
    hp_ref[...] = h_ref[...] * dinv
    dinv_ref[...] = dinv


def _tc_scale(cnt, h):
    return pl.pallas_call(
        _scale_body,
        out_shape=[
            jax.ShapeDtypeStruct((N, D), jnp.float32),
            jax.ShapeDtypeStruct((N, 1), jnp.float32),
        ],
    )(cnt, h)


def _mid_body(agg_ref, hprev_ref, dinv_ref, b_ref, w_ref, h_ref, hp_ref):
    dinv = dinv_ref[...]
    z = (dinv * (agg_ref[0] + agg_ref[1])
         + (2.0 * dinv * dinv) * hprev_ref[...] + b_ref[...])
    h = jnp.dot(z, w_ref[...], preferred_element_type=jnp.float32)
    h_ref[...] = h
    hp_ref[...] = h * dinv


def _tc_mid(agg, hprev, dinv, b2d, W):
    return pl.pallas_call(
        _mid_body,
        out_shape=[
            jax.ShapeDtypeStruct((N, D), jnp.float32),
            jax.ShapeDtypeStruct((N, D), jnp.float32),
        ],
    )(agg, hprev, dinv, b2d, W)


def _final_body(agg_ref, hprev_ref, dinv_ref, b_ref, out_ref):
    dinv = dinv_ref[...]
    out_ref[...] = (dinv * (agg_ref[0] + agg_ref[1])
                    + (2.0 * dinv * dinv) * hprev_ref[...] + b_ref[...])


def _tc_final(agg, hprev, dinv, b2d):
    return pl.pallas_call(
        _final_body,
        out_shape=jax.ShapeDtypeStruct((N, D), jnp.float32),
    )(agg, hprev, dinv, b2d)


def kernel(x, edge_index, W1, b1, W2, b2):
    col = edge_index[1]
    zeros = jnp.zeros((RPT, D), jnp.float32)
    b1d = b1.reshape(1, D)
    b2d = b2.reshape(1, D)

    h = _tc_mm1(x, W1)
    cnt = _sc_hist(col, zeros)
    hp, dinv = _tc_scale(cnt, h)
    for b in (b1d, b2d, b2d, b2d):
        agg = _sc_agg(hp, edge_index, zeros)
        h, hp = _tc_mid(agg, h, dinv, b, W2)
    agg = _sc_agg(hp, edge_index, zeros)
    return _tc_final(agg, h, dinv, b2d)
